# baseline (speedup 1.0000x reference)
import functools

import jax
import jax.numpy as jnp
from jax import lax
from jax.experimental import pallas as pl
from jax.experimental.pallas import tpu as pltpu
from jax.experimental.pallas import tpu_sc as plsc

B = 16
N_NODES = 100000
N_PAD = 100096
NC = 2
NS = 16
NW = NC * NS
G = 128
S = 3
GPB = 12
GPW = 792
NBLK = GPW // GPB
E_PAD = NW * GPW * G
ROWS_PER_TILE = N_PAD // NS
SCALE = float(1 << 20)


def _sc_body(pp_hbm, pn_hbm, src_hbm, dst_hbm, zero_hbm, out_hbm,
             idx_s, idx_d,
             ra0, rb0, rc0, rd0, ra1, rb1, rc1, rd1, ra2, rb2, rc2, rd2,
             acc,
             sem_is, sem_id,
             ga0, gb0, gc0, gd0, ga1, gb1, gc1, gd1, ga2, gb2, gc2, gd2,
             sa0, sb0, sc0, sd0, sa1, sb1, sc1, sd1, sa2, sb2, sc2, sd2):
    RA = [ra0, ra1, ra2]
    RB = [rb0, rb1, rb2]
    RC = [rc0, rc1, rc2]
    RD = [rd0, rd1, rd2]
    GA = [ga0, ga1, ga2]
    GB = [gb0, gb1, gb2]
    GC = [gc0, gc1, gc2]
    GD = [gd0, gd1, gd2]
    SA = [sa0, sa1, sa2]
    SB = [sb0, sb1, sb2]
    SC = [sc0, sc1, sc2]
    SD = [sd0, sd1, sd2]

    c = lax.axis_index("c")
    s = lax.axis_index("s")
    wid = s * NC + c
    row_base = wid * GPW

    def idx_row(arr, j):
        return arr.at[lax.rem(lax.div(j, GPB), 2), lax.rem(j, GPB)]

    stripe = pl.ds(s * ROWS_PER_TILE, ROWS_PER_TILE)
    pltpu.sync_copy(zero_hbm.at[stripe], acc.at[stripe])

    pltpu.sync_copy(src_hbm.at[pl.ds(row_base, GPB)], idx_s.at[0])
    pltpu.sync_copy(dst_hbm.at[pl.ds(row_base, GPB)], idx_d.at[0])
    plsc.subcore_barrier()

    def issue_gathers(j, t):
        pltpu.async_copy(pp_hbm.at[idx_row(idx_s, j)], RA[t], GA[t])
        pltpu.async_copy(pn_hbm.at[idx_row(idx_d, j)], RB[t], GB[t])
        pltpu.async_copy(pn_hbm.at[idx_row(idx_s, j)], RC[t], GC[t])
        pltpu.async_copy(pp_hbm.at[idx_row(idx_d, j)], RD[t], GD[t])

    for t in range(S):
        issue_gathers(t, t)

    @pl.loop(0, GPW // S)
    def _k(k):
        for t in range(S):
            j = S * k + t
            jm = j - S

            @pl.when(j >= S)
            def _():
                pltpu.make_async_copy(
                    RA[t], acc.at[idx_row(idx_s, jm)], SA[t]).wait()
                pltpu.make_async_copy(
                    RB[t], acc.at[idx_row(idx_s, jm)], SB[t]).wait()
                pltpu.make_async_copy(
                    RC[t], acc.at[idx_row(idx_d, jm)], SC[t]).wait()
                pltpu.make_async_copy(
                    RD[t], acc.at[idx_row(idx_d, jm)], SD[t]).wait()

            blk = lax.div(j, GPB)

            @pl.when((lax.rem(j, GPB) == S) & (blk + 1 < NBLK))
            def _():
                nb = blk + 1
                npar = lax.rem(nb, 2)
                pltpu.async_copy(
                    src_hbm.at[pl.ds(row_base + nb * GPB, GPB)],
                    idx_s.at[npar], sem_is)
                pltpu.async_copy(
                    dst_hbm.at[pl.ds(row_base + nb * GPB, GPB)],
                    idx_d.at[npar], sem_id)

            pltpu.make_async_copy(
                pp_hbm.at[idx_row(idx_s, j)], RA[t], GA[t]).wait()
            pltpu.make_async_copy(
                pn_hbm.at[idx_row(idx_d, j)], RB[t], GB[t]).wait()
            pltpu.make_async_copy(
                pn_hbm.at[idx_row(idx_s, j)], RC[t], GC[t]).wait()
            pltpu.make_async_copy(
                pp_hbm.at[idx_row(idx_d, j)], RD[t], GD[t]).wait()
            pltpu.async_copy(RA[t], acc.at[idx_row(idx_s, j)], SA[t],
                             add=True)
            pltpu.async_copy(RB[t], acc.at[idx_row(idx_s, j)], SB[t],
                             add=True)
            pltpu.async_copy(RC[t], acc.at[idx_row(idx_d, j)], SC[t],
                             add=True)
            pltpu.async_copy(RD[t], acc.at[idx_row(idx_d, j)], SD[t],
                             add=True)

            jn = j + S

            @pl.when(jn < GPW)
            def _():
                @pl.when(lax.rem(jn, GPB) == 0)
                def _():
                    nb2 = lax.div(jn, GPB)
                    npar2 = lax.rem(nb2, 2)
                    pltpu.make_async_copy(
                        src_hbm.at[pl.ds(row_base + nb2 * GPB, GPB)],
                        idx_s.at[npar2], sem_is).wait()
                    pltpu.make_async_copy(
                        dst_hbm.at[pl.ds(row_base + nb2 * GPB, GPB)],
                        idx_d.at[npar2], sem_id).wait()
                issue_gathers(jn, t)

    for t in range(S):
        j = GPW - S + t
        pltpu.make_async_copy(RA[t], acc.at[idx_row(idx_s, j)], SA[t]).wait()
        pltpu.make_async_copy(RB[t], acc.at[idx_row(idx_s, j)], SB[t]).wait()
        pltpu.make_async_copy(RC[t], acc.at[idx_row(idx_d, j)], SC[t]).wait()
        pltpu.make_async_copy(RD[t], acc.at[idx_row(idx_d, j)], SD[t]).wait()

    plsc.subcore_barrier()
    pltpu.sync_copy(acc.at[stripe], out_hbm.at[c, stripe])


@jax.jit
def _laplacian_sc(pp, pn, src2d, dst2d, zero):
    mesh = plsc.VectorSubcoreMesh(
        core_axis_name="c", subcore_axis_name="s",
        num_cores=NC, num_subcores=NS)
    row_bufs = []
    for _ in range(S):
        row_bufs += [pltpu.VMEM((G, B), jnp.int32)] * 4
    f = functools.partial(
        pl.kernel,
        out_type=jax.ShapeDtypeStruct((NC, N_PAD, B), jnp.int32),
        mesh=mesh,
        scratch_types=[
            pltpu.VMEM((2, GPB, G), jnp.int32),
            pltpu.VMEM((2, GPB, G), jnp.int32),
        ] + row_bufs + [
            pltpu.VMEM_SHARED((N_PAD, B), jnp.int32),
        ] + [pltpu.SemaphoreType.DMA] * 26,
        compiler_params=pltpu.CompilerParams(use_tc_tiling_on_sc=False),
    )(_sc_body)
    return f(pp, pn, src2d, dst2d, zero)


def kernel(p, edge_src, edge_dst):
    n_edges = edge_src.shape[0]
    pad = E_PAD - n_edges
    pt = jnp.pad(p.T, ((0, N_PAD - N_NODES), (0, 0)))
    pp = jnp.rint(pt * SCALE).astype(jnp.int32)
    pn = -pp
    src2d = jnp.concatenate(
        [edge_src, jnp.zeros((pad,), edge_src.dtype)]).reshape(-1, G)
    dst2d = jnp.concatenate(
        [edge_dst, jnp.zeros((pad,), edge_dst.dtype)]).reshape(-1, G)
    zero = jnp.zeros((N_PAD, B), jnp.int32)
    parts = _laplacian_sc(pp, pn, src2d, dst2d, zero)
    acc = parts[0, :N_NODES] + parts[1, :N_NODES]
    return (acc.astype(jnp.float32) * (1.0 / SCALE)).T


# speedup vs baseline: 1.2457x; 1.2457x over previous
import functools

import jax
import jax.numpy as jnp
from jax import lax
from jax.experimental import pallas as pl
from jax.experimental.pallas import tpu as pltpu
from jax.experimental.pallas import tpu_sc as plsc

B = 16
N_NODES = 100000
N_PAD = 100096
NC = 2
NS = 16
NW = NC * NS
G = 128
S = 3
GPB = 12
GPW = 792
NBLK = GPW // GPB
E_PAD = NW * GPW * G
ROWS_PER_TILE = N_PAD // NS


def _sc_body(pt_hbm, src_hbm, dst_hbm, zero_hbm, out_hbm,
             idx_s, idx_d,
             rs0, rd0, g0, ng0, rs1, rd1, g1, ng1, rs2, rd2, g2, ng2,
             acc,
             sem_is, sem_id,
             gs0, gd0, gs1, gd1, gs2, gd2,
             ss0, sd0, ss1, sd1, ss2, sd2):
    RS = [rs0, rs1, rs2]
    RD = [rd0, rd1, rd2]
    GG = [g0, g1, g2]
    NG = [ng0, ng1, ng2]
    GS = [gs0, gs1, gs2]
    GD = [gd0, gd1, gd2]
    SS = [ss0, ss1, ss2]
    SD = [sd0, sd1, sd2]

    c = lax.axis_index("c")
    s = lax.axis_index("s")
    wid = s * NC + c
    row_base = wid * GPW

    def idx_row(arr, j):
        return arr.at[lax.rem(lax.div(j, GPB), 2), lax.rem(j, GPB)]

    stripe = pl.ds(s * ROWS_PER_TILE, ROWS_PER_TILE)
    pltpu.sync_copy(zero_hbm.at[stripe], acc.at[stripe])

    pltpu.sync_copy(src_hbm.at[pl.ds(row_base, GPB)], idx_s.at[0])
    pltpu.sync_copy(dst_hbm.at[pl.ds(row_base, GPB)], idx_d.at[0])
    plsc.subcore_barrier()
    for t in range(S):
        pltpu.async_copy(pt_hbm.at[idx_s.at[0, t]], RS[t], GS[t])
        pltpu.async_copy(pt_hbm.at[idx_d.at[0, t]], RD[t], GD[t])

    def compute(rs, rd, g, ng):
        for r in range(G):
            a = rs[r]
            d = rd[r]
            g[r] = a - d
            ng[r] = d - a

    @pl.loop(0, GPW // S)
    def _k(k):
        for t in range(S):
            j = S * k + t
            jm = j - S

            @pl.when(j >= S)
            def _():
                pltpu.make_async_copy(
                    GG[t], acc.at[idx_row(idx_s, jm)], SS[t]).wait()
                pltpu.make_async_copy(
                    NG[t], acc.at[idx_row(idx_d, jm)], SD[t]).wait()

            blk = lax.div(j, GPB)

            @pl.when((lax.rem(j, GPB) == S) & (blk + 1 < NBLK))
            def _():
                nb = blk + 1
                npar = lax.rem(nb, 2)
                pltpu.async_copy(
                    src_hbm.at[pl.ds(row_base + nb * GPB, GPB)],
                    idx_s.at[npar], sem_is)
                pltpu.async_copy(
                    dst_hbm.at[pl.ds(row_base + nb * GPB, GPB)],
                    idx_d.at[npar], sem_id)

            pltpu.make_async_copy(
                pt_hbm.at[idx_row(idx_s, j)], RS[t], GS[t]).wait()
            pltpu.make_async_copy(
                pt_hbm.at[idx_row(idx_d, j)], RD[t], GD[t]).wait()
            compute(RS[t], RD[t], GG[t], NG[t])
            pltpu.async_copy(GG[t], acc.at[idx_row(idx_s, j)], SS[t],
                             add=True)
            pltpu.async_copy(NG[t], acc.at[idx_row(idx_d, j)], SD[t],
                             add=True)

            jn = j + S

            @pl.when(jn < GPW)
            def _():
                @pl.when(lax.rem(jn, GPB) == 0)
                def _():
                    nb2 = lax.div(jn, GPB)
                    npar2 = lax.rem(nb2, 2)
                    pltpu.make_async_copy(
                        src_hbm.at[pl.ds(row_base + nb2 * GPB, GPB)],
                        idx_s.at[npar2], sem_is).wait()
                    pltpu.make_async_copy(
                        dst_hbm.at[pl.ds(row_base + nb2 * GPB, GPB)],
                        idx_d.at[npar2], sem_id).wait()
                pltpu.async_copy(pt_hbm.at[idx_row(idx_s, jn)], RS[t], GS[t])
                pltpu.async_copy(pt_hbm.at[idx_row(idx_d, jn)], RD[t], GD[t])

    for t in range(S):
        j = GPW - S + t
        pltpu.make_async_copy(GG[t], acc.at[idx_row(idx_s, j)], SS[t]).wait()
        pltpu.make_async_copy(NG[t], acc.at[idx_row(idx_d, j)], SD[t]).wait()

    plsc.subcore_barrier()
    pltpu.sync_copy(acc.at[stripe], out_hbm.at[c, stripe])


@jax.jit
def _laplacian_sc(pt, src2d, dst2d, zero):
    mesh = plsc.VectorSubcoreMesh(
        core_axis_name="c", subcore_axis_name="s",
        num_cores=NC, num_subcores=NS)
    row_bufs = []
    for _ in range(S):
        row_bufs += [pltpu.VMEM((G, B), jnp.float32)] * 4
    f = functools.partial(
        pl.kernel,
        out_type=jax.ShapeDtypeStruct((NC, N_PAD, B), jnp.float32),
        mesh=mesh,
        scratch_types=[
            pltpu.VMEM((2, GPB, G), jnp.int32),
            pltpu.VMEM((2, GPB, G), jnp.int32),
        ] + row_bufs + [
            pltpu.VMEM_SHARED((N_PAD, B), jnp.float32),
        ] + [pltpu.SemaphoreType.DMA] * 14,
        compiler_params=pltpu.CompilerParams(use_tc_tiling_on_sc=False),
    )(_sc_body)
    return f(pt, src2d, dst2d, zero)


def kernel(p, edge_src, edge_dst):
    n_edges = edge_src.shape[0]
    pad = E_PAD - n_edges
    pt = jnp.pad(p.T, ((0, N_PAD - N_NODES), (0, 0)))
    src2d = jnp.concatenate(
        [edge_src, jnp.zeros((pad,), edge_src.dtype)]).reshape(-1, G)
    dst2d = jnp.concatenate(
        [edge_dst, jnp.zeros((pad,), edge_dst.dtype)]).reshape(-1, G)
    zero = jnp.zeros((N_PAD, B), jnp.float32)
    parts = _laplacian_sc(pt, src2d, dst2d, zero)
    return (parts[0, :N_NODES] + parts[1, :N_NODES]).T


# speedup vs baseline: 1.3733x; 1.1025x over previous
import functools

import jax
import jax.numpy as jnp
from jax import lax
from jax.experimental import pallas as pl
from jax.experimental.pallas import tpu as pltpu
from jax.experimental.pallas import tpu_sc as plsc

B = 16
N_NODES = 100000
N_PAD = 100096
NC = 2
NS = 16
G = 128
S = 3
GPB = 12
GPW0 = 1152
GPW1 = 432
NBLK0 = GPW0 // GPB
NBLK1 = GPW1 // GPB
E_PAD = NS * (GPW0 + GPW1) * G
ROWS_PER_TILE = N_PAD // NS


def _sc_body(pt_hbm, src_hbm, dst_hbm, zero_hbm, out_hbm,
             idx_s, idx_d,
             rs0, rd0, g0, ng0, rs1, rd1, g1, ng1, rs2, rd2, g2, ng2,
             acc,
             sem_is, sem_id,
             gs0, gd0, gs1, gd1, gs2, gd2,
             ss0, sd0, ss1, sd1, ss2, sd2):
    RS = [rs0, rs1, rs2]
    RD = [rd0, rd1, rd2]
    GG = [g0, g1, g2]
    NG = [ng0, ng1, ng2]
    GS = [gs0, gs1, gs2]
    GD = [gd0, gd1, gd2]
    SS = [ss0, ss1, ss2]
    SD = [sd0, sd1, sd2]

    c = lax.axis_index("c")
    s = lax.axis_index("s")
    gpw = jnp.where(c == 0, GPW0, GPW1)
    nblk = jnp.where(c == 0, NBLK0, NBLK1)
    row_base = jnp.where(c == 0, s * GPW0, NS * GPW0 + s * GPW1)

    def idx_row(arr, j):
        return arr.at[lax.rem(lax.div(j, GPB), 2), lax.rem(j, GPB)]

    stripe = pl.ds(s * ROWS_PER_TILE, ROWS_PER_TILE)
    pltpu.sync_copy(zero_hbm.at[stripe], acc.at[stripe])

    pltpu.sync_copy(src_hbm.at[pl.ds(row_base, GPB)], idx_s.at[0])
    pltpu.sync_copy(dst_hbm.at[pl.ds(row_base, GPB)], idx_d.at[0])
    plsc.subcore_barrier()
    for t in range(S):
        pltpu.async_copy(pt_hbm.at[idx_s.at[0, t]], RS[t], GS[t])
        pltpu.async_copy(pt_hbm.at[idx_d.at[0, t]], RD[t], GD[t])

    def compute(rs, rd, g, ng):
        for r in range(G):
            a = rs[r]
            d = rd[r]
            g[r] = a - d
            ng[r] = d - a

    @pl.loop(0, GPW0 // S)
    def _k(k):
        for t in range(S):
            j = S * k + t
            jm = j - S

            @pl.when(j < gpw)
            def _():
                @pl.when(j >= S)
                def _():
                    pltpu.make_async_copy(
                        GG[t], acc.at[idx_row(idx_s, jm)], SS[t]).wait()
                    pltpu.make_async_copy(
                        NG[t], acc.at[idx_row(idx_d, jm)], SD[t]).wait()

                blk = lax.div(j, GPB)

                @pl.when((lax.rem(j, GPB) == S) & (blk + 1 < nblk))
                def _():
                    nb = blk + 1
                    npar = lax.rem(nb, 2)
                    pltpu.async_copy(
                        src_hbm.at[pl.ds(row_base + nb * GPB, GPB)],
                        idx_s.at[npar], sem_is)
                    pltpu.async_copy(
                        dst_hbm.at[pl.ds(row_base + nb * GPB, GPB)],
                        idx_d.at[npar], sem_id)

                pltpu.make_async_copy(
                    pt_hbm.at[idx_row(idx_s, j)], RS[t], GS[t]).wait()
                pltpu.make_async_copy(
                    pt_hbm.at[idx_row(idx_d, j)], RD[t], GD[t]).wait()
                compute(RS[t], RD[t], GG[t], NG[t])
                pltpu.async_copy(GG[t], acc.at[idx_row(idx_s, j)], SS[t],
                                 add=True)
                pltpu.async_copy(NG[t], acc.at[idx_row(idx_d, j)], SD[t],
                                 add=True)

                jn = j + S

                @pl.when(jn < gpw)
                def _():
                    @pl.when(lax.rem(jn, GPB) == 0)
                    def _():
                        nb2 = lax.div(jn, GPB)
                        npar2 = lax.rem(nb2, 2)
                        pltpu.make_async_copy(
                            src_hbm.at[pl.ds(row_base + nb2 * GPB, GPB)],
                            idx_s.at[npar2], sem_is).wait()
                        pltpu.make_async_copy(
                            dst_hbm.at[pl.ds(row_base + nb2 * GPB, GPB)],
                            idx_d.at[npar2], sem_id).wait()
                    pltpu.async_copy(
                        pt_hbm.at[idx_row(idx_s, jn)], RS[t], GS[t])
                    pltpu.async_copy(
                        pt_hbm.at[idx_row(idx_d, jn)], RD[t], GD[t])

    for t in range(S):
        j = gpw - S + t
        pltpu.make_async_copy(GG[t], acc.at[idx_row(idx_s, j)], SS[t]).wait()
        pltpu.make_async_copy(NG[t], acc.at[idx_row(idx_d, j)], SD[t]).wait()

    plsc.subcore_barrier()
    pltpu.sync_copy(acc.at[stripe], out_hbm.at[c, stripe])


@jax.jit
def _laplacian_sc(pt, src2d, dst2d, zero):
    mesh = plsc.VectorSubcoreMesh(
        core_axis_name="c", subcore_axis_name="s",
        num_cores=NC, num_subcores=NS)
    row_bufs = []
    for _ in range(S):
        row_bufs += [pltpu.VMEM((G, B), jnp.float32)] * 4
    f = functools.partial(
        pl.kernel,
        out_type=jax.ShapeDtypeStruct((NC, N_PAD, B), jnp.float32),
        mesh=mesh,
        scratch_types=[
            pltpu.VMEM((2, GPB, G), jnp.int32),
            pltpu.VMEM((2, GPB, G), jnp.int32),
        ] + row_bufs + [
            pltpu.VMEM_SHARED((N_PAD, B), jnp.float32),
        ] + [pltpu.SemaphoreType.DMA] * 14,
        compiler_params=pltpu.CompilerParams(use_tc_tiling_on_sc=False),
    )(_sc_body)
    return f(pt, src2d, dst2d, zero)


def kernel(p, edge_src, edge_dst):
    n_edges = edge_src.shape[0]
    pad = E_PAD - n_edges
    pt = jnp.pad(p.T, ((0, N_PAD - N_NODES), (0, 0)))
    src2d = jnp.concatenate(
        [edge_src, jnp.zeros((pad,), edge_src.dtype)]).reshape(-1, G)
    dst2d = jnp.concatenate(
        [edge_dst, jnp.zeros((pad,), edge_dst.dtype)]).reshape(-1, G)
    zero = jnp.zeros((N_PAD, B), jnp.float32)
    parts = _laplacian_sc(pt, src2d, dst2d, zero)
    return (parts[0, :N_NODES] + parts[1, :N_NODES]).T


# speedup vs baseline: 1.4255x; 1.0380x over previous
import functools

import jax
import jax.numpy as jnp
from jax import lax
from jax.experimental import pallas as pl
from jax.experimental.pallas import tpu as pltpu
from jax.experimental.pallas import tpu_sc as plsc

B = 16
N_NODES = 100000
N_PAD = 100096
NC = 2
NS = 16
G = 128
S = 3
GPB = 12
GPW0 = 1344
GPW1 = 240
NBLK0 = GPW0 // GPB
NBLK1 = GPW1 // GPB
E_PAD = NS * (GPW0 + GPW1) * G
ROWS_PER_TILE = N_PAD // NS


def _sc_body(pt_hbm, src_hbm, dst_hbm, zero_hbm, out_hbm,
             idx_s, idx_d,
             rs0, rd0, g0, ng0, rs1, rd1, g1, ng1, rs2, rd2, g2, ng2,
             acc,
             sem_is, sem_id,
             gs0, gd0, gs1, gd1, gs2, gd2,
             ss0, sd0, ss1, sd1, ss2, sd2):
    RS = [rs0, rs1, rs2]
    RD = [rd0, rd1, rd2]
    GG = [g0, g1, g2]
    NG = [ng0, ng1, ng2]
    GS = [gs0, gs1, gs2]
    GD = [gd0, gd1, gd2]
    SS = [ss0, ss1, ss2]
    SD = [sd0, sd1, sd2]

    c = lax.axis_index("c")
    s = lax.axis_index("s")
    gpw = jnp.where(c == 0, GPW0, GPW1)
    nblk = jnp.where(c == 0, NBLK0, NBLK1)
    row_base = jnp.where(c == 0, s * GPW0, NS * GPW0 + s * GPW1)

    def idx_row(arr, j):
        return arr.at[lax.rem(lax.div(j, GPB), 2), lax.rem(j, GPB)]

    stripe = pl.ds(s * ROWS_PER_TILE, ROWS_PER_TILE)
    pltpu.sync_copy(zero_hbm.at[stripe], acc.at[stripe])

    pltpu.sync_copy(src_hbm.at[pl.ds(row_base, GPB)], idx_s.at[0])
    pltpu.sync_copy(dst_hbm.at[pl.ds(row_base, GPB)], idx_d.at[0])
    plsc.subcore_barrier()
    for t in range(S):
        pltpu.async_copy(pt_hbm.at[idx_s.at[0, t]], RS[t], GS[t])
        pltpu.async_copy(pt_hbm.at[idx_d.at[0, t]], RD[t], GD[t])

    def compute(rs, rd, g, ng):
        for r in range(G):
            a = rs[r]
            d = rd[r]
            g[r] = a - d
            ng[r] = d - a

    @pl.loop(0, GPW0 // S)
    def _k(k):
        for t in range(S):
            j = S * k + t
            jm = j - S

            @pl.when(j < gpw)
            def _():
                @pl.when(j >= S)
                def _():
                    pltpu.make_async_copy(
                        GG[t], acc.at[idx_row(idx_s, jm)], SS[t]).wait()
                    pltpu.make_async_copy(
                        NG[t], acc.at[idx_row(idx_d, jm)], SD[t]).wait()

                blk = lax.div(j, GPB)

                @pl.when((lax.rem(j, GPB) == S) & (blk + 1 < nblk))
                def _():
                    nb = blk + 1
                    npar = lax.rem(nb, 2)
                    pltpu.async_copy(
                        src_hbm.at[pl.ds(row_base + nb * GPB, GPB)],
                        idx_s.at[npar], sem_is)
                    pltpu.async_copy(
                        dst_hbm.at[pl.ds(row_base + nb * GPB, GPB)],
                        idx_d.at[npar], sem_id)

                pltpu.make_async_copy(
                    pt_hbm.at[idx_row(idx_s, j)], RS[t], GS[t]).wait()
                pltpu.make_async_copy(
                    pt_hbm.at[idx_row(idx_d, j)], RD[t], GD[t]).wait()
                compute(RS[t], RD[t], GG[t], NG[t])
                pltpu.async_copy(GG[t], acc.at[idx_row(idx_s, j)], SS[t],
                                 add=True)
                pltpu.async_copy(NG[t], acc.at[idx_row(idx_d, j)], SD[t],
                                 add=True)

                jn = j + S

                @pl.when(jn < gpw)
                def _():
                    @pl.when(lax.rem(jn, GPB) == 0)
                    def _():
                        nb2 = lax.div(jn, GPB)
                        npar2 = lax.rem(nb2, 2)
                        pltpu.make_async_copy(
                            src_hbm.at[pl.ds(row_base + nb2 * GPB, GPB)],
                            idx_s.at[npar2], sem_is).wait()
                        pltpu.make_async_copy(
                            dst_hbm.at[pl.ds(row_base + nb2 * GPB, GPB)],
                            idx_d.at[npar2], sem_id).wait()
                    pltpu.async_copy(
                        pt_hbm.at[idx_row(idx_s, jn)], RS[t], GS[t])
                    pltpu.async_copy(
                        pt_hbm.at[idx_row(idx_d, jn)], RD[t], GD[t])

    for t in range(S):
        j = gpw - S + t
        pltpu.make_async_copy(GG[t], acc.at[idx_row(idx_s, j)], SS[t]).wait()
        pltpu.make_async_copy(NG[t], acc.at[idx_row(idx_d, j)], SD[t]).wait()

    plsc.subcore_barrier()
    pltpu.sync_copy(acc.at[stripe], out_hbm.at[c, stripe])


@jax.jit
def _laplacian_sc(pt, src2d, dst2d, zero):
    mesh = plsc.VectorSubcoreMesh(
        core_axis_name="c", subcore_axis_name="s",
        num_cores=NC, num_subcores=NS)
    row_bufs = []
    for _ in range(S):
        row_bufs += [pltpu.VMEM((G, B), jnp.float32)] * 4
    f = functools.partial(
        pl.kernel,
        out_type=jax.ShapeDtypeStruct((NC, N_PAD, B), jnp.float32),
        mesh=mesh,
        scratch_types=[
            pltpu.VMEM((2, GPB, G), jnp.int32),
            pltpu.VMEM((2, GPB, G), jnp.int32),
        ] + row_bufs + [
            pltpu.VMEM_SHARED((N_PAD, B), jnp.float32),
        ] + [pltpu.SemaphoreType.DMA] * 14,
        compiler_params=pltpu.CompilerParams(use_tc_tiling_on_sc=False),
    )(_sc_body)
    return f(pt, src2d, dst2d, zero)


def kernel(p, edge_src, edge_dst):
    n_edges = edge_src.shape[0]
    pad = E_PAD - n_edges
    pt = jnp.pad(p.T, ((0, N_PAD - N_NODES), (0, 0)))
    src2d = jnp.concatenate(
        [edge_src, jnp.zeros((pad,), edge_src.dtype)]).reshape(-1, G)
    dst2d = jnp.concatenate(
        [edge_dst, jnp.zeros((pad,), edge_dst.dtype)]).reshape(-1, G)
    zero = jnp.zeros((N_PAD, B), jnp.float32)
    parts = _laplacian_sc(pt, src2d, dst2d, zero)
    return (parts[0, :N_NODES] + parts[1, :N_NODES]).T


# speedup vs baseline: 1.4581x; 1.0229x over previous
import functools

import jax
import jax.numpy as jnp
from jax import lax
from jax.experimental import pallas as pl
from jax.experimental.pallas import tpu as pltpu
from jax.experimental.pallas import tpu_sc as plsc

B = 16
N_NODES = 100000
N_PAD = 100096
NC = 2
NS = 16
G = 128
S = 3
GPB = 12
GPW0 = 1344
GPW1 = 240
NBLK0 = GPW0 // GPB
NBLK1 = GPW1 // GPB
E_PAD = NS * (GPW0 + GPW1) * G
ROWS_PER_TILE = N_PAD // NS


def _sc_body(pt_hbm, src_hbm, dst_hbm, zero_hbm, out_hbm,
             idx_s, idx_d,
             rs0, rd0, g0, ng0, rs1, rd1, g1, ng1, rs2, rd2, g2, ng2,
             acc,
             sem_is, sem_id,
             gs0, gd0, gs1, gd1, gs2, gd2,
             ss0, sd0, ss1, sd1, ss2, sd2):
    RS = [rs0, rs1, rs2]
    RD = [rd0, rd1, rd2]
    GG = [g0, g1, g2]
    NG = [ng0, ng1, ng2]
    GS = [gs0, gs1, gs2]
    GD = [gd0, gd1, gd2]
    SS = [ss0, ss1, ss2]
    SD = [sd0, sd1, sd2]

    c = lax.axis_index("c")
    s = lax.axis_index("s")
    gpw = jnp.where(c == 0, GPW0, GPW1)
    nblk = jnp.where(c == 0, NBLK0, NBLK1)
    row_base = jnp.where(c == 0, s * GPW0, NS * GPW0 + s * GPW1)

    def idx_row(arr, j):
        return arr.at[lax.rem(lax.div(j, GPB), 2), lax.rem(j, GPB)]

    stripe = pl.ds(s * ROWS_PER_TILE, ROWS_PER_TILE)
    pltpu.sync_copy(zero_hbm.at[stripe], acc.at[stripe])

    pltpu.sync_copy(src_hbm.at[pl.ds(row_base, GPB)], idx_s.at[0])
    pltpu.sync_copy(dst_hbm.at[pl.ds(row_base, GPB)], idx_d.at[0])
    plsc.subcore_barrier()
    for t in range(S):
        pltpu.async_copy(pt_hbm.at[idx_s.at[0, t]], RS[t], GS[t])
        pltpu.async_copy(pt_hbm.at[idx_d.at[0, t]], RD[t], GD[t])

    def compute(rs, rd, g, ng):
        for r in range(G):
            a = rs[r]
            d = rd[r]
            g[r] = a - d
            ng[r] = d - a

    @pl.loop(0, GPW0 // S)
    def _k(k):
        for t in range(S):
            j = S * k + t
            jm = j - S

            @pl.when(j < gpw)
            def _():
                @pl.when(j >= S)
                def _():
                    pltpu.make_async_copy(
                        GG[t], acc.at[idx_row(idx_s, jm)], SS[t]).wait()
                    pltpu.make_async_copy(
                        NG[t], acc.at[idx_row(idx_d, jm)], SD[t]).wait()

                blk = lax.div(j, GPB)

                @pl.when((lax.rem(j, GPB) == S) & (blk + 1 < nblk))
                def _():
                    nb = blk + 1
                    npar = lax.rem(nb, 2)
                    pltpu.async_copy(
                        src_hbm.at[pl.ds(row_base + nb * GPB, GPB)],
                        idx_s.at[npar], sem_is)
                    pltpu.async_copy(
                        dst_hbm.at[pl.ds(row_base + nb * GPB, GPB)],
                        idx_d.at[npar], sem_id)

                pltpu.make_async_copy(
                    pt_hbm.at[idx_row(idx_s, j)], RS[t], GS[t]).wait()
                pltpu.make_async_copy(
                    pt_hbm.at[idx_row(idx_d, j)], RD[t], GD[t]).wait()
                compute(RS[t], RD[t], GG[t], NG[t])
                pltpu.async_copy(GG[t], acc.at[idx_row(idx_s, j)], SS[t],
                                 add=True)
                pltpu.async_copy(NG[t], acc.at[idx_row(idx_d, j)], SD[t],
                                 add=True)

                jn = j + S

                @pl.when(jn < gpw)
                def _():
                    @pl.when(lax.rem(jn, GPB) == 0)
                    def _():
                        nb2 = lax.div(jn, GPB)
                        npar2 = lax.rem(nb2, 2)
                        pltpu.make_async_copy(
                            src_hbm.at[pl.ds(row_base + nb2 * GPB, GPB)],
                            idx_s.at[npar2], sem_is).wait()
                        pltpu.make_async_copy(
                            dst_hbm.at[pl.ds(row_base + nb2 * GPB, GPB)],
                            idx_d.at[npar2], sem_id).wait()
                    pltpu.async_copy(
                        pt_hbm.at[idx_row(idx_s, jn)], RS[t], GS[t])
                    pltpu.async_copy(
                        pt_hbm.at[idx_row(idx_d, jn)], RD[t], GD[t])

    for t in range(S):
        j = gpw - S + t
        pltpu.make_async_copy(GG[t], acc.at[idx_row(idx_s, j)], SS[t]).wait()
        pltpu.make_async_copy(NG[t], acc.at[idx_row(idx_d, j)], SD[t]).wait()

    plsc.subcore_barrier()
    pltpu.sync_copy(acc.at[stripe], out_hbm.at[c, stripe])


@jax.jit
def _laplacian_sc(pt, src2d, dst2d, zero):
    mesh = plsc.VectorSubcoreMesh(
        core_axis_name="c", subcore_axis_name="s",
        num_cores=NC, num_subcores=NS)
    row_bufs = []
    for _ in range(S):
        row_bufs += [pltpu.VMEM((G, B), jnp.float32)] * 4
    f = functools.partial(
        pl.kernel,
        out_type=jax.ShapeDtypeStruct((NC, N_PAD, B), jnp.float32),
        mesh=mesh,
        scratch_types=[
            pltpu.VMEM((2, GPB, G), jnp.int32),
            pltpu.VMEM((2, GPB, G), jnp.int32),
        ] + row_bufs + [
            pltpu.VMEM_SHARED((N_PAD, B), jnp.float32),
        ] + [pltpu.SemaphoreType.DMA] * 14,
        compiler_params=pltpu.CompilerParams(use_tc_tiling_on_sc=False),
    )(_sc_body)
    return f(pt, src2d, dst2d, zero)


BN = 4352


def _tc_epilogue_body(parts_ref, out_ref):
    x = parts_ref[0] + parts_ref[1]
    r = lax.broadcasted_iota(jnp.int32, (B, B), 0)
    co = lax.broadcasted_iota(jnp.int32, (B, B), 1)
    eye = (r == co).astype(jnp.float32)
    out_ref[...] = lax.dot_general(
        eye, x, (((1,), (1,)), ((), ())),
        preferred_element_type=jnp.float32)


@jax.jit
def _tc_epilogue(parts):
    return pl.pallas_call(
        _tc_epilogue_body,
        out_shape=jax.ShapeDtypeStruct((B, N_PAD), jnp.float32),
        grid=(N_PAD // BN,),
        in_specs=[pl.BlockSpec((2, BN, B), lambda i: (0, i, 0))],
        out_specs=pl.BlockSpec((B, BN), lambda i: (0, i)),
    )(parts)[:, :N_NODES]


def kernel(p, edge_src, edge_dst):
    n_edges = edge_src.shape[0]
    pad = E_PAD - n_edges
    pt = jnp.pad(p.T, ((0, N_PAD - N_NODES), (0, 0)))
    src2d = jnp.concatenate(
        [edge_src, jnp.zeros((pad,), edge_src.dtype)]).reshape(-1, G)
    dst2d = jnp.concatenate(
        [edge_dst, jnp.zeros((pad,), edge_dst.dtype)]).reshape(-1, G)
    zero = jnp.zeros((N_PAD, B), jnp.float32)
    parts = _laplacian_sc(pt, src2d, dst2d, zero)
    return _tc_epilogue(parts)


# speedup vs baseline: 1.4613x; 1.0022x over previous
import functools

import jax
import jax.numpy as jnp
from jax import lax
from jax.experimental import pallas as pl
from jax.experimental.pallas import tpu as pltpu
from jax.experimental.pallas import tpu_sc as plsc

B = 16
N_NODES = 100000
N_PAD = 100096
NC = 2
NS = 16
G = 128
S = 3
GPB = 12
GPW0 = 1344
GPW1 = 240
NBLK0 = GPW0 // GPB
NBLK1 = GPW1 // GPB
E_PAD = NS * (GPW0 + GPW1) * G
ROWS_PER_TILE = N_PAD // NS


def _sc_body(pt_hbm, src_hbm, dst_hbm, zero_hbm, out_hbm,
             idx_s, idx_d,
             rs0, rd0, g0, ng0, rs1, rd1, g1, ng1, rs2, rd2, g2, ng2,
             acc,
             sem_is, sem_id,
             gs0, gd0, gs1, gd1, gs2, gd2,
             ss0, sd0, ss1, sd1, ss2, sd2):
    RS = [rs0, rs1, rs2]
    RD = [rd0, rd1, rd2]
    GG = [g0, g1, g2]
    NG = [ng0, ng1, ng2]
    GS = [gs0, gs1, gs2]
    GD = [gd0, gd1, gd2]
    SS = [ss0, ss1, ss2]
    SD = [sd0, sd1, sd2]

    c = lax.axis_index("c")
    s = lax.axis_index("s")
    gpw = jnp.where(c == 0, GPW0, GPW1)
    nblk = jnp.where(c == 0, NBLK0, NBLK1)
    row_base = jnp.where(c == 0, s * GPW0, NS * GPW0 + s * GPW1)

    def idx_row(arr, j):
        return arr.at[lax.rem(lax.div(j, GPB), 2), lax.rem(j, GPB)]

    stripe = pl.ds(s * ROWS_PER_TILE, ROWS_PER_TILE)
    pltpu.sync_copy(zero_hbm.at[stripe], acc.at[stripe])

    pltpu.sync_copy(src_hbm.at[pl.ds(row_base, GPB)], idx_s.at[0])
    pltpu.sync_copy(dst_hbm.at[pl.ds(row_base, GPB)], idx_d.at[0])
    plsc.subcore_barrier()
    for t in range(S):
        pltpu.async_copy(pt_hbm.at[idx_s.at[0, t]], RS[t], GS[t])
        pltpu.async_copy(pt_hbm.at[idx_d.at[0, t]], RD[t], GD[t])

    def compute(rs, rd, g, ng):
        for r in range(G):
            a = rs[r]
            d = rd[r]
            g[r] = a - d
            ng[r] = d - a

    @pl.loop(0, GPW0 // S)
    def _k(k):
        for t in range(S):
            j = S * k + t
            jm = j - S

            @pl.when(j < gpw)
            def _():
                @pl.when(j >= S)
                def _():
                    pltpu.make_async_copy(
                        GG[t], acc.at[idx_row(idx_s, jm)], SS[t]).wait()
                    pltpu.make_async_copy(
                        NG[t], acc.at[idx_row(idx_d, jm)], SD[t]).wait()

                blk = lax.div(j, GPB)

                @pl.when((lax.rem(j, GPB) == S) & (blk + 1 < nblk))
                def _():
                    nb = blk + 1
                    npar = lax.rem(nb, 2)
                    pltpu.async_copy(
                        src_hbm.at[pl.ds(row_base + nb * GPB, GPB)],
                        idx_s.at[npar], sem_is)
                    pltpu.async_copy(
                        dst_hbm.at[pl.ds(row_base + nb * GPB, GPB)],
                        idx_d.at[npar], sem_id)

                pltpu.make_async_copy(
                    pt_hbm.at[idx_row(idx_s, j)], RS[t], GS[t]).wait()
                pltpu.make_async_copy(
                    pt_hbm.at[idx_row(idx_d, j)], RD[t], GD[t]).wait()
                compute(RS[t], RD[t], GG[t], NG[t])
                pltpu.async_copy(GG[t], acc.at[idx_row(idx_s, j)], SS[t],
                                 add=True)
                pltpu.async_copy(NG[t], acc.at[idx_row(idx_d, j)], SD[t],
                                 add=True)

                jn = j + S

                @pl.when(jn < gpw)
                def _():
                    @pl.when(lax.rem(jn, GPB) == 0)
                    def _():
                        nb2 = lax.div(jn, GPB)
                        npar2 = lax.rem(nb2, 2)
                        pltpu.make_async_copy(
                            src_hbm.at[pl.ds(row_base + nb2 * GPB, GPB)],
                            idx_s.at[npar2], sem_is).wait()
                        pltpu.make_async_copy(
                            dst_hbm.at[pl.ds(row_base + nb2 * GPB, GPB)],
                            idx_d.at[npar2], sem_id).wait()
                    pltpu.async_copy(
                        pt_hbm.at[idx_row(idx_s, jn)], RS[t], GS[t])
                    pltpu.async_copy(
                        pt_hbm.at[idx_row(idx_d, jn)], RD[t], GD[t])

    for t in range(S):
        j = gpw - S + t
        pltpu.make_async_copy(GG[t], acc.at[idx_row(idx_s, j)], SS[t]).wait()
        pltpu.make_async_copy(NG[t], acc.at[idx_row(idx_d, j)], SD[t]).wait()

    plsc.subcore_barrier()
    pltpu.sync_copy(acc.at[stripe], out_hbm.at[c, stripe])


@jax.jit
def _laplacian_sc(pt, src2d, dst2d, zero):
    mesh = plsc.VectorSubcoreMesh(
        core_axis_name="c", subcore_axis_name="s",
        num_cores=NC, num_subcores=NS)
    row_bufs = []
    for _ in range(S):
        row_bufs += [pltpu.VMEM((G, B), jnp.float32)] * 4
    f = functools.partial(
        pl.kernel,
        out_type=jax.ShapeDtypeStruct((NC, N_PAD, B), jnp.float32),
        mesh=mesh,
        scratch_types=[
            pltpu.VMEM((2, GPB, G), jnp.int32),
            pltpu.VMEM((2, GPB, G), jnp.int32),
        ] + row_bufs + [
            pltpu.VMEM_SHARED((N_PAD, B), jnp.float32),
        ] + [pltpu.SemaphoreType.DMA] * 14,
        compiler_params=pltpu.CompilerParams(use_tc_tiling_on_sc=False),
    )(_sc_body)
    return f(pt, src2d, dst2d, zero)


BN = 4352


def _tc_epilogue_body(parts_ref, out_ref):
    x = parts_ref[0] + parts_ref[1]
    out_ref[...] = x.T


@jax.jit
def _tc_epilogue(parts):
    return pl.pallas_call(
        _tc_epilogue_body,
        out_shape=jax.ShapeDtypeStruct((B, N_PAD), jnp.float32),
        grid=(N_PAD // BN,),
        in_specs=[pl.BlockSpec((2, BN, B), lambda i: (0, i, 0))],
        out_specs=pl.BlockSpec((B, BN), lambda i: (0, i)),
    )(parts)[:, :N_NODES]


def kernel(p, edge_src, edge_dst):
    n_edges = edge_src.shape[0]
    pad = E_PAD - n_edges
    pt = jnp.pad(p.T, ((0, N_PAD - N_NODES), (0, 0)))
    src2d = jnp.concatenate(
        [edge_src, jnp.zeros((pad,), edge_src.dtype)]).reshape(-1, G)
    dst2d = jnp.concatenate(
        [edge_dst, jnp.zeros((pad,), edge_dst.dtype)]).reshape(-1, G)
    zero = jnp.zeros((N_PAD, B), jnp.float32)
    parts = _laplacian_sc(pt, src2d, dst2d, zero)
    return _tc_epilogue(parts)
